# baseline (speedup 1.0000x reference)
import functools

import jax
import jax.numpy as jnp
from jax import lax
from jax.experimental import pallas as pl
from jax.experimental.pallas import tpu as pltpu
from jax.experimental.pallas import tpu_sc as plsc

N_NODES = 10000
N_EDGES = 160000
D_FEAT = 256
NUM_HEADS = 8
CH = 128
NCHUNK = N_EDGES // CH
OROW = NCHUNK * NUM_HEADS


def _tc_table(x, W2, b2):
    M = x.shape[0]
    BM = 2000
    H2 = 2 * NUM_HEADS

    def body(x_ref, w_ref, b_ref, t_ref):
        t_ref[...] = (
            jnp.dot(x_ref[...], w_ref[...], preferred_element_type=jnp.float32)
            + b_ref[...]
        )

    return pl.pallas_call(
        body,
        grid=(M // BM,),
        in_specs=[
            pl.BlockSpec((BM, D_FEAT), lambda i: (i, 0)),
            pl.BlockSpec((D_FEAT, H2), lambda i: (0, 0)),
            pl.BlockSpec((1, H2), lambda i: (0, 0)),
        ],
        out_specs=pl.BlockSpec((BM, H2), lambda i: (i, 0)),
        out_shape=jax.ShapeDtypeStruct((M, H2), jnp.float32),
    )(x, W2, b2)


def _sc_edge_combine(T, ei):
    info = plsc.get_sparse_core_info()
    NW = info.num_cores * info.num_subcores
    CPW = 40
    LASTN = NCHUNK - (NW - 1) * CPW
    mesh = plsc.VectorSubcoreMesh(core_axis_name="c", subcore_axis_name="s")

    @functools.partial(
        pl.kernel,
        out_type=jax.ShapeDtypeStruct((OROW, CH), jnp.float32),
        mesh=mesh,
        compiler_params=pltpu.CompilerParams(
            use_tc_tiling_on_sc=False, needs_layout_passes=False
        ),
        scratch_types=[
            pltpu.VMEM((CPW, CH), jnp.int32),
            pltpu.VMEM((CPW, CH), jnp.int32),
            pltpu.VMEM((CH, 16), jnp.float32),
            pltpu.VMEM((CH, 16), jnp.float32),
            pltpu.VMEM((CH, 16), jnp.float32),
            pltpu.VMEM((CH, 16), jnp.float32),
            pltpu.VMEM((16, 129), jnp.float32),
            pltpu.VMEM((16, 129), jnp.float32),
            pltpu.SemaphoreType.DMA,
            pltpu.SemaphoreType.DMA,
            pltpu.SemaphoreType.DMA,
            pltpu.SemaphoreType.DMA,
            pltpu.SemaphoreType.DMA,
            pltpu.SemaphoreType.DMA,
        ],
    )
    def k(t_hbm, e_hbm, out_hbm, io, idd, a0, a1, b0, b1,
          o0, o1, sa0, sa1, sb0, sb1, so0, so1):
        wid = lax.axis_index("s") * info.num_cores + lax.axis_index("c")
        base_c = wid * CPW
        is_last = wid == NW - 1
        n = jnp.where(is_last, LASTN, CPW)

        @pl.when(jnp.logical_not(is_last))
        def _():
            pltpu.sync_copy(e_hbm.at[pl.ds(base_c, CPW)], io)
            pltpu.sync_copy(e_hbm.at[pl.ds(NCHUNK + base_c, CPW)], idd)

        @pl.when(is_last)
        def _():
            pltpu.sync_copy(
                e_hbm.at[pl.ds(base_c, LASTN)], io.at[pl.ds(0, LASTN)]
            )
            pltpu.sync_copy(
                e_hbm.at[pl.ds(NCHUNK + base_c, LASTN)], idd.at[pl.ds(0, LASTN)]
            )

        A = [a0, a1]
        B = [b0, b1]
        O = [o0, o1]
        SA = [sa0, sa1]
        SB = [sb0, sb1]
        SO = [so0, so1]

        iota = lax.iota(jnp.int32, 16)
        iotax = lax.bitwise_xor(iota, 8)

        def start_gathers(jj, p):
            pltpu.make_async_copy(t_hbm.at[io.at[jj]], A[p], SA[p]).start()
            pltpu.make_async_copy(t_hbm.at[idd.at[jj]], B[p], SB[p]).start()

        def wait_gathers(p):
            pltpu.make_async_copy(t_hbm.at[io.at[0]], A[p], SA[p]).wait()
            pltpu.make_async_copy(t_hbm.at[idd.at[0]], B[p], SB[p]).wait()

        def start_out(jj, p):
            r0 = pl.multiple_of((base_c + jj) * NUM_HEADS, NUM_HEADS)
            pltpu.make_async_copy(
                O[p].at[pl.ds(0, NUM_HEADS), pl.ds(0, CH)],
                out_hbm.at[pl.ds(r0, NUM_HEADS)],
                SO[p],
            ).start()

        def wait_out(p):
            pltpu.make_async_copy(
                O[p].at[pl.ds(0, NUM_HEADS), pl.ds(0, CH)],
                out_hbm.at[pl.ds(0, NUM_HEADS)],
                SO[p],
            ).wait()

        def compute(p):
            for i in range(CH):
                ci = jnp.full((16,), i, jnp.int32)
                plsc.store_scatter(O[p], [iota, ci], A[p][i, :])
                plsc.addupdate_scatter(O[p], [iotax, ci], B[p][i, :])

        start_gathers(0, 0)
        start_gathers(1, 1)

        def body(t, _):
            j0 = t * 2
            for p in (0, 1):
                jj = j0 + p
                wait_gathers(p)

                @pl.when(t > 0)
                def _():
                    wait_out(p)

                compute(p)
                start_out(jj, p)

                @pl.when(jj + 2 < n)
                def _():
                    start_gathers(jj + 2, p)

            return 0

        lax.fori_loop(0, n // 2, body, 0)
        wait_out(0)
        wait_out(1)

    return k(T, ei)


def kernel(x, edge_index, W, b):
    W2 = jnp.concatenate([W[:D_FEAT], W[D_FEAT:]], axis=1)
    b2 = jnp.concatenate([b, jnp.zeros((NUM_HEADS,), jnp.float32)])
    T = _tc_table(x, W2, b2.reshape(1, 2 * NUM_HEADS))

    ei = edge_index.astype(jnp.int32).reshape(2 * NCHUNK, CH)
    out2 = _sc_edge_combine(T, ei)
    return (
        out2.reshape(NCHUNK, NUM_HEADS, CH)
        .transpose(0, 2, 1)
        .reshape(N_EDGES, NUM_HEADS)
    )


# speedup vs baseline: 1.2877x; 1.2877x over previous
import functools

import jax
import jax.numpy as jnp
from jax import lax
from jax.experimental import pallas as pl
from jax.experimental.pallas import tpu as pltpu
from jax.experimental.pallas import tpu_sc as plsc

N_NODES = 10000
N_EDGES = 160000
D_FEAT = 256
NUM_HEADS = 8
CH = 128
NCHUNK = N_EDGES // CH
OROW = NCHUNK * NUM_HEADS


def _tc_table(x, W2, b2):
    M = x.shape[0]
    BM = 2000
    H2 = 2 * NUM_HEADS

    def body(x_ref, w_ref, b_ref, t_ref):
        t_ref[...] = (
            jnp.dot(x_ref[...], w_ref[...], preferred_element_type=jnp.float32)
            + b_ref[...]
        )

    return pl.pallas_call(
        body,
        grid=(M // BM,),
        in_specs=[
            pl.BlockSpec((BM, D_FEAT), lambda i: (i, 0)),
            pl.BlockSpec((D_FEAT, H2), lambda i: (0, 0)),
            pl.BlockSpec((1, H2), lambda i: (0, 0)),
        ],
        out_specs=pl.BlockSpec((BM, H2), lambda i: (i, 0)),
        out_shape=jax.ShapeDtypeStruct((M, H2), jnp.float32),
    )(x, W2, b2)


def _sc_edge_combine(T, ei):
    info = plsc.get_sparse_core_info()
    NW = info.num_cores * info.num_subcores
    CPW = 40
    LASTN = NCHUNK - (NW - 1) * CPW
    mesh = plsc.VectorSubcoreMesh(core_axis_name="c", subcore_axis_name="s")

    @functools.partial(
        pl.kernel,
        out_type=jax.ShapeDtypeStruct((OROW, CH), jnp.float32),
        mesh=mesh,
        compiler_params=pltpu.CompilerParams(
            use_tc_tiling_on_sc=False, needs_layout_passes=False
        ),
        scratch_types=[
            pltpu.VMEM((CPW, CH), jnp.int32),
            pltpu.VMEM((CPW, CH), jnp.int32),
            pltpu.VMEM((CH, 16), jnp.float32),
            pltpu.VMEM((CH, 16), jnp.float32),
            pltpu.VMEM((CH, 16), jnp.float32),
            pltpu.VMEM((CH, 16), jnp.float32),
            pltpu.VMEM((NUM_HEADS, CH), jnp.float32),
            pltpu.VMEM((NUM_HEADS, CH), jnp.float32),
            pltpu.SemaphoreType.DMA,
            pltpu.SemaphoreType.DMA,
            pltpu.SemaphoreType.DMA,
            pltpu.SemaphoreType.DMA,
            pltpu.SemaphoreType.DMA,
            pltpu.SemaphoreType.DMA,
        ],
    )
    def k(t_hbm, e_hbm, out_hbm, io, idd, a0, a1, b0, b1,
          o0, o1, sa0, sa1, sb0, sb1, so0, so1):
        wid = lax.axis_index("s") * info.num_cores + lax.axis_index("c")
        base_c = wid * CPW
        is_last = wid == NW - 1
        n = jnp.where(is_last, LASTN, CPW)

        @pl.when(jnp.logical_not(is_last))
        def _():
            pltpu.sync_copy(e_hbm.at[pl.ds(base_c, CPW)], io)
            pltpu.sync_copy(e_hbm.at[pl.ds(NCHUNK + base_c, CPW)], idd)

        @pl.when(is_last)
        def _():
            pltpu.sync_copy(
                e_hbm.at[pl.ds(base_c, LASTN)], io.at[pl.ds(0, LASTN)]
            )
            pltpu.sync_copy(
                e_hbm.at[pl.ds(NCHUNK + base_c, LASTN)], idd.at[pl.ds(0, LASTN)]
            )

        A = [a0, a1]
        B = [b0, b1]
        O = [o0, o1]
        SA = [sa0, sa1]
        SB = [sb0, sb1]
        SO = [so0, so1]

        iota = lax.iota(jnp.int32, 16)
        iotax = lax.bitwise_xor(iota, 8)

        def start_gathers(jj, p):
            pltpu.make_async_copy(t_hbm.at[io.at[jj]], A[p], SA[p]).start()
            pltpu.make_async_copy(t_hbm.at[idd.at[jj]], B[p], SB[p]).start()

        def wait_gathers(p):
            pltpu.make_async_copy(t_hbm.at[io.at[0]], A[p], SA[p]).wait()
            pltpu.make_async_copy(t_hbm.at[idd.at[0]], B[p], SB[p]).wait()

        def start_out(jj, p):
            r0 = pl.multiple_of((base_c + jj) * NUM_HEADS, NUM_HEADS)
            pltpu.make_async_copy(
                O[p], out_hbm.at[pl.ds(r0, NUM_HEADS)], SO[p]
            ).start()

        def wait_out(p):
            pltpu.make_async_copy(
                O[p], out_hbm.at[pl.ds(0, NUM_HEADS)], SO[p]
            ).wait()

        def compute(p):
            for h in range(NUM_HEADS):
                ch = jnp.full((16,), h, jnp.int32)
                ch8 = jnp.full((16,), h + 8, jnp.int32)
                for kk in range(CH // 16):
                    ridx = iota + (kk * 16)
                    va = plsc.load_gather(A[p], [ridx, ch])
                    vb = plsc.load_gather(B[p], [ridx, ch8])
                    O[p][h, pl.ds(kk * 16, 16)] = va + vb

        start_gathers(0, 0)
        start_gathers(1, 1)

        def body(t, _):
            j0 = t * 2
            for p in (0, 1):
                jj = j0 + p
                wait_gathers(p)

                @pl.when(t > 0)
                def _():
                    wait_out(p)

                compute(p)
                start_out(jj, p)

                @pl.when(jj + 2 < n)
                def _():
                    start_gathers(jj + 2, p)

            return 0

        lax.fori_loop(0, n // 2, body, 0)
        wait_out(0)
        wait_out(1)

    return k(T, ei)


def kernel(x, edge_index, W, b):
    W2 = jnp.concatenate([W[:D_FEAT], W[D_FEAT:]], axis=1)
    b2 = jnp.concatenate([b, jnp.zeros((NUM_HEADS,), jnp.float32)])
    T = _tc_table(x, W2, b2.reshape(1, 2 * NUM_HEADS))

    ei = edge_index.astype(jnp.int32).reshape(2 * NCHUNK, CH)
    out2 = _sc_edge_combine(T, ei)
    return (
        out2.reshape(NCHUNK, NUM_HEADS, CH)
        .transpose(0, 2, 1)
        .reshape(N_EDGES, NUM_HEADS)
    )
